# Initial kernel scaffold; baseline (speedup 1.0000x reference)
import jax
import jax.numpy as jnp
from jax.experimental import pallas as pl


def kernel(patient_features, corpus_embeddings, W, b, top_k):
    raise NotImplementedError("write your pallas kernel here")


# speedup vs baseline: 1.3063x; 1.3063x over previous
import functools

import jax
import jax.numpy as jnp
from jax import lax
from jax.experimental import pallas as pl
from jax.experimental.pallas import tpu as pltpu
from jax.experimental.pallas import tpu_sc as plsc

B = 1024
N_DOCS = 100000
EMB = 64
FIN = 32
K = 32
TILE = 2000
N_TILES = 50
CW = K + TILE


def _topk_body(pf_ref, w_ref, b_ref, corpus_ref, vals_ref, idx_ref,
               pemb, curv, curi):
    t = pl.program_id(0)

    @pl.when(t == 0)
    def _init():
        pe = lax.dot_general(pf_ref[...].astype(jnp.bfloat16),
                             w_ref[...].astype(jnp.bfloat16),
                             (((1,), (1,)), ((), ())),
                             preferred_element_type=jnp.float32) + b_ref[...]
        nrm = jnp.sqrt(jnp.sum(pe * pe, axis=1, keepdims=True))
        pemb[...] = pe / jnp.maximum(nrm, 1e-12)
        curv[...] = jnp.full((B, K), -jnp.inf, dtype=jnp.float32)
        curi[...] = jnp.zeros((B, K), dtype=jnp.int32)

    c = corpus_ref[...]
    cn = jnp.sqrt(jnp.sum(c * c, axis=1, keepdims=True))
    cnorm = c / jnp.maximum(cn, 1e-12)
    scores = lax.dot_general(pemb[...].astype(jnp.bfloat16),
                             cnorm.astype(jnp.bfloat16),
                             (((1,), (1,)), ((), ())),
                             preferred_element_type=jnp.float32)
    col = lax.broadcasted_iota(jnp.int32, (B, TILE), 1)
    doc = col + t * TILE

    work = jnp.concatenate([curv[...], scores], axis=1)
    wid = jnp.concatenate([curi[...], doc], axis=1)
    lane = lax.broadcasted_iota(jnp.int32, (B, CW), 1)
    vs, ixs = [], []
    for _ in range(K):
        m = jnp.max(work, axis=1, keepdims=True)
        ismax = work == m
        pos = jnp.min(jnp.where(ismax, lane, CW), axis=1, keepdims=True)
        sel = lane == pos
        pidx = jnp.sum(jnp.where(sel, wid, 0), axis=1, keepdims=True)
        vs.append(m)
        ixs.append(pidx)
        work = jnp.where(sel, -jnp.inf, work)
    curv[...] = jnp.concatenate(vs, axis=1)
    curi[...] = jnp.concatenate(ixs, axis=1)

    @pl.when(t == pl.num_programs(0) - 1)
    def _fin():
        vals_ref[...] = curv[...]
        idx_ref[...] = curi[...]


_topk_call = pl.pallas_call(
    _topk_body,
    grid=(N_TILES,),
    in_specs=[
        pl.BlockSpec((B, FIN), lambda i: (0, 0)),
        pl.BlockSpec((EMB, FIN), lambda i: (0, 0)),
        pl.BlockSpec((1, EMB), lambda i: (0, 0)),
        pl.BlockSpec((TILE, EMB), lambda i: (i, 0)),
    ],
    out_specs=[
        pl.BlockSpec((B, K), lambda i: (0, 0)),
        pl.BlockSpec((B, K), lambda i: (0, 0)),
    ],
    out_shape=[
        jax.ShapeDtypeStruct((B, K), jnp.float32),
        jax.ShapeDtypeStruct((B, K), jnp.int32),
    ],
    scratch_shapes=[
        pltpu.VMEM((B, EMB), jnp.float32),
        pltpu.VMEM((B, K), jnp.float32),
        pltpu.VMEM((B, K), jnp.int32),
    ],
)

NW = 32
GB = B * K
BPW = GB // NW
D_PAD = 128
CHUNK = 512


def _gather_body(table_hbm, idx_hbm, out_hbm, idx_v, rows_v, sem):
    wid = lax.axis_index("s") * 2 + lax.axis_index("c")
    base = wid * BPW
    for c in range(BPW // CHUNK):
        off = base + c * CHUNK
        pltpu.sync_copy(idx_hbm.at[pl.ds(off, CHUNK)], idx_v)
        pltpu.async_copy(table_hbm.at[idx_v], rows_v, sem).wait()
        pltpu.sync_copy(rows_v, out_hbm.at[pl.ds(off, CHUNK)])


@functools.lru_cache(maxsize=1)
def _make_gather():
    return functools.partial(
        pl.kernel,
        mesh=plsc.VectorSubcoreMesh(core_axis_name="c", subcore_axis_name="s"),
        out_type=jax.ShapeDtypeStruct((GB, D_PAD), jnp.float32),
        scratch_types=[
            pltpu.VMEM((CHUNK,), jnp.int32),
            pltpu.VMEM((CHUNK, D_PAD), jnp.float32),
            pltpu.SemaphoreType.DMA,
        ],
    )(_gather_body)


def kernel(patient_features, corpus_embeddings, W, b, top_k):
    vals, idxs = _topk_call(patient_features, W, b.reshape(1, EMB),
                            corpus_embeddings)
    table = jnp.pad(corpus_embeddings, ((0, 0), (0, D_PAD - EMB)))
    flat = _make_gather()(table, idxs.reshape(GB))
    return flat[:, :EMB].reshape(B, K, EMB), vals, idxs


# speedup vs baseline: 10.4743x; 8.0183x over previous
import jax
import jax.numpy as jnp
from jax import lax
from jax.experimental import pallas as pl
from jax.experimental.pallas import tpu as pltpu
from jax.experimental.pallas import tpu_sc as plsc

B = 1024
N_DOCS = 100000
EMB = 64
FIN = 32
K = 32
TILE = 2048
N_TILES = 50
N_PAD = N_TILES * TILE
NBLK = N_PAD // 128
BLK_SEL = 48
CAND = BLK_SEL * 128
NSUB = CAND // 16
SUB_SEL = 40
RB = 256
NEG = float("-inf")


def _stream_body(pf_ref, w_ref, b_ref, corpus_ref, s3_ref, cm_ref, pemb):
    t = pl.program_id(0)

    @pl.when(t == 0)
    def _init():
        pe = lax.dot_general(pf_ref[...].astype(jnp.bfloat16),
                             w_ref[...].astype(jnp.bfloat16),
                             (((1,), (1,)), ((), ())),
                             preferred_element_type=jnp.float32) + b_ref[...]
        nrm = jnp.sqrt(jnp.sum(pe * pe, axis=1, keepdims=True))
        pemb[...] = pe / jnp.maximum(nrm, 1e-12)

    c = corpus_ref[...]
    cn = jnp.sqrt(jnp.sum(c * c, axis=1, keepdims=True))
    cnorm = c / jnp.maximum(cn, 1e-12)
    scores = lax.dot_general(pemb[...].astype(jnp.bfloat16),
                             cnorm.astype(jnp.bfloat16),
                             (((1,), (1,)), ((), ())),
                             preferred_element_type=jnp.float32)

    def emit(sc):
        cms = []
        for m in range(TILE // 128):
            sl = sc[:, 128 * m:128 * (m + 1)]
            s3_ref[:, m, :] = sl
            cms.append(jnp.max(sl, axis=1, keepdims=True))
        pad = jnp.full((B, 128 - TILE // 128), NEG, dtype=jnp.float32)
        cm_ref[...] = jnp.concatenate(cms + [pad], axis=1)

    emit(scores)

    @pl.when((t + 1) * TILE > N_DOCS)
    def _mask_tail():
        vc = N_DOCS - t * TILE
        pos = lax.broadcasted_iota(jnp.int32, (B, TILE), 1)
        emit(jnp.where(pos < vc, scores, NEG))


_k1 = pl.pallas_call(
    _stream_body,
    grid=(N_TILES,),
    in_specs=[
        pl.BlockSpec((B, FIN), lambda i: (0, 0)),
        pl.BlockSpec((EMB, FIN), lambda i: (0, 0)),
        pl.BlockSpec((1, EMB), lambda i: (0, 0)),
        pl.BlockSpec((TILE, EMB), lambda i: (i, 0)),
    ],
    out_specs=[
        pl.BlockSpec((B, TILE // 128, 128), lambda i: (0, i, 0)),
        pl.BlockSpec((B, 128), lambda i: (0, i)),
    ],
    out_shape=[
        jax.ShapeDtypeStruct((B, NBLK, 128), jnp.float32),
        jax.ShapeDtypeStruct((B, N_TILES * 128), jnp.float32),
    ],
    scratch_shapes=[pltpu.VMEM((B, EMB), jnp.float32)],
)


def _sel_body(cm_ref, sel_ref):
    parts = [cm_ref[:, 128 * t:128 * t + (TILE // 128)] for t in range(N_TILES)]
    L = jnp.concatenate(parts, axis=1)
    lane = lax.broadcasted_iota(jnp.int32, (B, NBLK), 1)
    cols = []
    for _ in range(BLK_SEL):
        a = jnp.argmax(L, axis=1).reshape(B, 1)
        cols.append(a)
        L = jnp.where(lane == a, NEG, L)
    sel_ref[...] = jnp.concatenate(cols, axis=1)


_k15 = pl.pallas_call(
    _sel_body,
    in_specs=[pl.BlockSpec((B, N_TILES * 128), lambda: (0, 0))],
    out_specs=pl.BlockSpec((B, BLK_SEL), lambda: (0, 0)),
    out_shape=jax.ShapeDtypeStruct((B, BLK_SEL), jnp.int32),
)


def _fin_body(cand_ref, sel_ref, vals_ref, idx_ref):
    M = cand_ref[:, 0, :]
    for m in range(1, BLK_SEL):
        M = jnp.maximum(M, cand_ref[:, m, :])
    lane = lax.broadcasted_iota(jnp.int32, (RB, 128), 1)
    scs = []
    for _ in range(SUB_SEL):
        a = jnp.argmax(M, axis=1).reshape(RB, 1)
        scs.append(a)
        M = jnp.where(lane == a, NEG, M)
    sc_ids = jnp.concatenate(scs, axis=1)
    cs_parts, md_parts = [], []
    for m in range(BLK_SEL):
        cs_parts.append(jnp.take_along_axis(cand_ref[:, m, :], sc_ids, axis=1))
        md_parts.append(sel_ref[:, m:m + 1] * 128 + sc_ids)
    CS = jnp.concatenate(cs_parts, axis=1)
    MD = jnp.concatenate(md_parts, axis=1)
    BIG = jnp.int32(2**30)
    vs, ds = [], []
    for _ in range(K):
        v = jnp.max(CS, axis=1, keepdims=True)
        tie = CS == v
        pick = jnp.min(jnp.where(tie, MD, BIG), axis=1, keepdims=True)
        vs.append(v)
        ds.append(pick)
        CS = jnp.where(MD == pick, NEG, CS)
    vals_ref[...] = jnp.concatenate(vs, axis=1)
    idx_ref[...] = jnp.concatenate(ds, axis=1)


_k3 = pl.pallas_call(
    _fin_body,
    grid=(B // RB,),
    in_specs=[
        pl.BlockSpec((RB, BLK_SEL, 128), lambda i: (i, 0, 0)),
        pl.BlockSpec((RB, BLK_SEL), lambda i: (i, 0)),
    ],
    out_specs=[
        pl.BlockSpec((RB, K), lambda i: (i, 0)),
        pl.BlockSpec((RB, K), lambda i: (i, 0)),
    ],
    out_shape=[
        jax.ShapeDtypeStruct((B, K), jnp.float32),
        jax.ShapeDtypeStruct((B, K), jnp.int32),
    ],
)


def _make_sc_gather(n_rows, width, chunk):
    nw = 32
    bpw = n_rows // nw

    def body(table_hbm, idx_hbm, out_hbm, idx_v, rows_v, sem):
        wid = lax.axis_index("s") * 2 + lax.axis_index("c")
        base = wid * bpw
        for c in range(bpw // chunk):
            off = base + c * chunk
            pltpu.sync_copy(idx_hbm.at[pl.ds(off, chunk)], idx_v)
            pltpu.async_copy(table_hbm.at[idx_v], rows_v, sem).wait()
            pltpu.sync_copy(rows_v, out_hbm.at[pl.ds(off, chunk)])

    return pl.kernel(
        body,
        mesh=plsc.VectorSubcoreMesh(core_axis_name="c", subcore_axis_name="s"),
        out_type=jax.ShapeDtypeStruct((n_rows, width), jnp.float32),
        scratch_types=[
            pltpu.VMEM((chunk,), jnp.int32),
            pltpu.VMEM((chunk, width), jnp.float32),
            pltpu.SemaphoreType.DMA,
        ],
    )


GB2 = B * BLK_SEL
GB = B * K
_gather_cache = {}


def _sc_gather(key, n_rows, width, chunk):
    if key not in _gather_cache:
        _gather_cache[key] = _make_sc_gather(n_rows, width, chunk)
    return _gather_cache[key]


def kernel(patient_features, corpus_embeddings, W, b, top_k):
    corpus_pad = jnp.pad(corpus_embeddings, ((0, N_PAD - N_DOCS), (0, 0)))
    scores3, cmwide = _k1(patient_features, W, b.reshape(1, EMB), corpus_pad)
    sel = _k15(cmwide)
    qidx = (jnp.arange(B, dtype=jnp.int32)[:, None] * NBLK + sel).reshape(GB2)
    cand = _sc_gather("k2", GB2, 128, 512)(scores3.reshape(B * NBLK, 128), qidx)
    vals, idxs = _k3(cand.reshape(B, BLK_SEL, 128), sel)
    table = jnp.pad(corpus_embeddings, ((0, 0), (0, 128 - EMB)))
    flat = _sc_gather("k4", GB, 128, 512)(table, idxs.reshape(GB))
    return flat[:, :EMB].reshape(B, K, EMB), vals, idxs


# speedup vs baseline: 11.4416x; 1.0923x over previous
import jax
import jax.numpy as jnp
from jax import lax
from jax.experimental import pallas as pl
from jax.experimental.pallas import tpu as pltpu
from jax.experimental.pallas import tpu_sc as plsc

B = 1024
N_DOCS = 100000
EMB = 64
FIN = 32
K = 32
TILE = 2048
N_TILES = 50
N_PAD = N_TILES * TILE
NBLK = N_PAD // 128
BLK_SEL = 48
SUB_SEL = 40
GSEL = 40
RB = 256
NEG = float("-inf")


def _stream_body(pf_ref, w_ref, b_ref, corpus_ref, s3_ref, cm_ref, pemb):
    t = pl.program_id(0)

    @pl.when(t == 0)
    def _init():
        pe = lax.dot_general(pf_ref[...].astype(jnp.bfloat16),
                             w_ref[...].astype(jnp.bfloat16),
                             (((1,), (1,)), ((), ())),
                             preferred_element_type=jnp.float32) + b_ref[...]
        nrm = jnp.sqrt(jnp.sum(pe * pe, axis=1, keepdims=True))
        pemb[...] = pe / jnp.maximum(nrm, 1e-12)

    c = corpus_ref[...][:, :EMB]
    cn = jnp.sqrt(jnp.sum(c * c, axis=1, keepdims=True))
    cnorm = c / jnp.maximum(cn, 1e-12)
    scores = lax.dot_general(pemb[...].astype(jnp.bfloat16),
                             cnorm.astype(jnp.bfloat16),
                             (((1,), (1,)), ((), ())),
                             preferred_element_type=jnp.float32)

    def emit(sc):
        cms = []
        for m in range(TILE // 128):
            sl = sc[:, 128 * m:128 * (m + 1)]
            s3_ref[:, m, :] = sl
            cms.append(jnp.max(sl, axis=1, keepdims=True))
        pad = jnp.full((B, 128 - TILE // 128), NEG, dtype=jnp.float32)
        cm_ref[...] = jnp.concatenate(cms + [pad], axis=1)

    emit(scores)

    @pl.when((t + 1) * TILE > N_DOCS)
    def _mask_tail():
        vc = N_DOCS - t * TILE
        pos = lax.broadcasted_iota(jnp.int32, (B, TILE), 1)
        emit(jnp.where(pos < vc, scores, NEG))


_k1 = pl.pallas_call(
    _stream_body,
    grid=(N_TILES,),
    in_specs=[
        pl.BlockSpec((B, FIN), lambda i: (0, 0)),
        pl.BlockSpec((EMB, FIN), lambda i: (0, 0)),
        pl.BlockSpec((1, EMB), lambda i: (0, 0)),
        pl.BlockSpec((TILE, 128), lambda i: (i, 0)),
    ],
    out_specs=[
        pl.BlockSpec((B, TILE // 128, 128), lambda i: (0, i, 0)),
        pl.BlockSpec((B, 128), lambda i: (0, i)),
    ],
    out_shape=[
        jax.ShapeDtypeStruct((B, NBLK, 128), jnp.float32),
        jax.ShapeDtypeStruct((B, N_TILES * 128), jnp.float32),
    ],
    scratch_shapes=[pltpu.VMEM((B, EMB), jnp.float32)],
)


def _sel_body(cm_ref, sel_ref):
    parts = [cm_ref[:, 128 * t:128 * t + (TILE // 128)] for t in range(N_TILES)]
    L = jnp.concatenate(parts, axis=1)
    lane = lax.broadcasted_iota(jnp.int32, (B, NBLK), 1)
    cols = []
    for _ in range(BLK_SEL):
        a = jnp.argmax(L, axis=1).reshape(B, 1)
        cols.append(a)
        L = jnp.where(lane == a, NEG, L)
    sel_ref[...] = jnp.concatenate(cols, axis=1)


_k15 = pl.pallas_call(
    _sel_body,
    in_specs=[pl.BlockSpec((B, N_TILES * 128), lambda: (0, 0))],
    out_specs=pl.BlockSpec((B, BLK_SEL), lambda: (0, 0)),
    out_shape=jax.ShapeDtypeStruct((B, BLK_SEL), jnp.int32),
)


def _fin_body(cand_ref, sel_ref, vals_ref, idx_ref):
    M = cand_ref[:, 0, :]
    for m in range(1, BLK_SEL):
        M = jnp.maximum(M, cand_ref[:, m, :])
    lane = lax.broadcasted_iota(jnp.int32, (RB, 128), 1)
    scs = []
    for _ in range(SUB_SEL):
        a = jnp.argmax(M, axis=1).reshape(RB, 1)
        scs.append(a)
        M = jnp.where(lane == a, NEG, M)
    sc_ids = jnp.concatenate(scs, axis=1)
    cs_parts, md_parts = [], []
    for m in range(BLK_SEL):
        cs_parts.append(jnp.take_along_axis(cand_ref[:, m, :], sc_ids, axis=1))
        md_parts.append(sel_ref[:, m:m + 1] * 128 + sc_ids)
    CS0 = jnp.concatenate(cs_parts, axis=1)
    MD0 = jnp.concatenate(md_parts, axis=1)
    BIG = jnp.int32(2**30)
    W0 = BLK_SEL * SUB_SEL
    GW = 128
    NG = W0 // GW
    G2 = CS0[:, :GW]
    for g in range(1, NG):
        G2 = jnp.maximum(G2, CS0[:, GW * g:GW * (g + 1)])
    lane2 = lax.broadcasted_iota(jnp.int32, (RB, GW), 1)
    g2s = []
    for _ in range(GSEL):
        a = jnp.argmax(G2, axis=1).reshape(RB, 1)
        g2s.append(a)
        G2 = jnp.where(lane2 == a, NEG, G2)
    g2_ids = jnp.concatenate(g2s, axis=1)
    cs2, md2 = [], []
    for g in range(NG):
        cs2.append(jnp.take_along_axis(CS0[:, GW * g:GW * (g + 1)], g2_ids, axis=1))
        md2.append(jnp.take_along_axis(MD0[:, GW * g:GW * (g + 1)], g2_ids, axis=1))
    CS = jnp.concatenate(cs2, axis=1)
    MD = jnp.concatenate(md2, axis=1)
    vs, ds = [], []
    for _ in range(K):
        v = jnp.max(CS, axis=1, keepdims=True)
        tie = CS == v
        pick = jnp.min(jnp.where(tie, MD, BIG), axis=1, keepdims=True)
        vs.append(v)
        ds.append(pick)
        CS = jnp.where(MD == pick, NEG, CS)
    vals_ref[...] = jnp.concatenate(vs, axis=1)
    idx_ref[...] = jnp.concatenate(ds, axis=1)


_k3 = pl.pallas_call(
    _fin_body,
    grid=(B // RB,),
    in_specs=[
        pl.BlockSpec((RB, BLK_SEL, 128), lambda i: (i, 0, 0)),
        pl.BlockSpec((RB, BLK_SEL), lambda i: (i, 0)),
    ],
    out_specs=[
        pl.BlockSpec((RB, K), lambda i: (i, 0)),
        pl.BlockSpec((RB, K), lambda i: (i, 0)),
    ],
    out_shape=[
        jax.ShapeDtypeStruct((B, K), jnp.float32),
        jax.ShapeDtypeStruct((B, K), jnp.int32),
    ],
)


def _make_sc_gather(n_rows, width, chunk):
    nw = 32
    bpw = n_rows // nw

    def body(table_hbm, idx_hbm, out_hbm, idx_v, rows_v, sem):
        wid = lax.axis_index("s") * 2 + lax.axis_index("c")
        base = wid * bpw
        for c in range(bpw // chunk):
            off = base + c * chunk
            pltpu.sync_copy(idx_hbm.at[pl.ds(off, chunk)], idx_v)
            pltpu.async_copy(table_hbm.at[idx_v], rows_v, sem).wait()
            pltpu.sync_copy(rows_v, out_hbm.at[pl.ds(off, chunk)])

    return pl.kernel(
        body,
        mesh=plsc.VectorSubcoreMesh(core_axis_name="c", subcore_axis_name="s"),
        out_type=jax.ShapeDtypeStruct((n_rows, width), jnp.float32),
        scratch_types=[
            pltpu.VMEM((chunk,), jnp.int32),
            pltpu.VMEM((chunk, width), jnp.float32),
            pltpu.SemaphoreType.DMA,
        ],
    )


GB2 = B * BLK_SEL
GB = B * K
_gather_cache = {}


def _sc_gather(key, n_rows, width, chunk):
    if key not in _gather_cache:
        _gather_cache[key] = _make_sc_gather(n_rows, width, chunk)
    return _gather_cache[key]


def kernel(patient_features, corpus_embeddings, W, b, top_k):
    corpus_pad = jnp.pad(corpus_embeddings,
                         ((0, N_PAD - N_DOCS), (0, 128 - EMB)))
    scores3, cmwide = _k1(patient_features, W, b.reshape(1, EMB), corpus_pad)
    sel = _k15(cmwide)
    qidx = (jnp.arange(B, dtype=jnp.int32)[:, None] * NBLK + sel).reshape(GB2)
    cand = _sc_gather("k2", GB2, 128, 512)(scores3.reshape(B * NBLK, 128), qidx)
    vals, idxs = _k3(cand.reshape(B, BLK_SEL, 128), sel)
    flat = _sc_gather("k4", GB, 128, 512)(corpus_pad, idxs.reshape(GB))
    return flat[:, :EMB].reshape(B, K, EMB), vals, idxs
